# baseline (speedup 1.0000x reference)
import functools

import jax
import jax.numpy as jnp
from jax import lax
from jax.experimental import pallas as pl
from jax.experimental.pallas import tpu as pltpu
from jax.experimental.pallas import tpu_sc as plsc

NC = 2
NS = 16
NW = NC * NS


def _lnorm(h, eps=1e-5):
    m = jnp.mean(h, axis=-1, keepdims=True)
    v = jnp.mean((h - m) ** 2, axis=-1, keepdims=True)
    return (h - m) * jax.lax.rsqrt(v + eps)


def _mlp_relu(h, w1, b1, w2, b2):
    h = _lnorm(h)
    h = jax.nn.relu(jnp.dot(h, w1, preferred_element_type=jnp.float32) + b1)
    h = _lnorm(h)
    h = jnp.dot(h, w2, preferred_element_type=jnp.float32) + b2
    return jax.nn.relu(h)


def _tc_first(x, w1, b1, w2, b2, *, rows):
    n, d = x.shape
    h = w1.shape[1]

    def body(x_ref, w1_ref, b1_ref, w2_ref, b2_ref, o_ref):
        o_ref[...] = _mlp_relu(x_ref[...], w1_ref[...], b1_ref[...],
                               w2_ref[...], b2_ref[...])

    full = lambda a: pl.BlockSpec(a.shape, lambda i: (0,) * a.ndim)
    return pl.pallas_call(
        body,
        grid=(n // rows,),
        in_specs=[pl.BlockSpec((rows, d), lambda i: (i, 0)),
                  full(w1), full(b1), full(w2), full(b2)],
        out_specs=pl.BlockSpec((rows, h), lambda i: (i, 0)),
        out_shape=jax.ShapeDtypeStruct((n, h), jnp.float32),
    )(x, w1, b1, w2, b2)


def _tc_mid(p, dw1, db1, dw2, db2, ew1, eb1, ew2, eb2, *, rows):
    _, n, h = p.shape

    def body(p_ref, dw1_ref, db1_ref, dw2_ref, db2_ref,
             ew1_ref, eb1_ref, ew2_ref, eb2_ref, o_ref):
        agg = p_ref[0]
        for t in range(1, p_ref.shape[0]):
            agg = agg + p_ref[t]
        hmid = _mlp_relu(agg, dw1_ref[...], db1_ref[...],
                         dw2_ref[...], db2_ref[...])
        o_ref[...] = _mlp_relu(hmid, ew1_ref[...], eb1_ref[...],
                               ew2_ref[...], eb2_ref[...])

    full = lambda a: pl.BlockSpec(a.shape, lambda i: (0,) * a.ndim)
    ws = [dw1, db1, dw2, db2, ew1, eb1, ew2, eb2]
    return pl.pallas_call(
        body,
        grid=(n // rows,),
        in_specs=[pl.BlockSpec((2, rows, h), lambda i: (0, i, 0))]
                 + [full(a) for a in ws],
        out_specs=pl.BlockSpec((rows, h), lambda i: (i, 0)),
        out_shape=jax.ShapeDtypeStruct((n, h), jnp.float32),
    )(p, *ws)


def _tc_final(p, dw1, db1, dw2, db2, cw1, cb1, cw2, cb2, *, rows):
    _, n, h = p.shape
    c = cw2.shape[1]

    def body(p_ref, dw1_ref, db1_ref, dw2_ref, db2_ref,
             cw1_ref, cb1_ref, cw2_ref, cb2_ref, o_ref):
        agg = p_ref[0]
        for t in range(1, p_ref.shape[0]):
            agg = agg + p_ref[t]
        hm = _mlp_relu(agg, dw1_ref[...], db1_ref[...],
                       dw2_ref[...], db2_ref[...])
        hm = jax.nn.relu(
            jnp.dot(hm, cw1_ref[...], preferred_element_type=jnp.float32)
            + cb1_ref[...])
        hm = _lnorm(hm)
        o_ref[...] = (jnp.dot(hm, cw2_ref[...],
                              preferred_element_type=jnp.float32)
                      + cb2_ref[...])

    full = lambda a: pl.BlockSpec(a.shape, lambda i: (0,) * a.ndim)
    ws = [dw1, db1, dw2, db2, cw1, cb1, cw2, cb2]
    return pl.pallas_call(
        body,
        grid=(n // rows,),
        in_specs=[pl.BlockSpec((2, rows, h), lambda i: (0, i, 0))]
                 + [full(a) for a in ws],
        out_specs=pl.BlockSpec((rows, c), lambda i: (i, 0)),
        out_shape=jax.ShapeDtypeStruct((n, c), jnp.float32),
    )(p, *ws)


@functools.partial(jax.jit, static_argnames=("b",))
def _sc_scatter(h, g0, s0, w0, g1, s1, w1, zeros, *, b):
    n, hdim = h.shape
    nb0 = g0.shape[1]
    nb1 = g1.shape[1]
    nch = hdim // 16
    assert nb0 % 4 == 0 and nb1 % 4 == 0 and nb1 >= 8 and b % 16 == 0
    rs = (n // NS) // 8 * 8
    tail = n - NS * rs

    mesh = plsc.VectorSubcoreMesh(core_axis_name="c", subcore_axis_name="s",
                                  num_cores=NC)

    @functools.partial(
        pl.kernel,
        mesh=mesh,
        compiler_params=pltpu.CompilerParams(use_tc_tiling_on_sc=False),
        out_type=jax.ShapeDtypeStruct((NC, n, hdim), jnp.float32),
        scratch_types=[
            pltpu.VMEM((nb0, b), jnp.int32),
            pltpu.VMEM((nb0, b), jnp.int32),
            pltpu.VMEM((nb0 * b,), jnp.float32),
            pltpu.VMEM((b, hdim), jnp.float32),
            pltpu.VMEM((b, hdim), jnp.float32),
            pltpu.VMEM((b, hdim), jnp.float32),
            pltpu.VMEM((b, hdim), jnp.float32),
            pltpu.VMEM_SHARED((n, hdim), jnp.float32),
            pltpu.SemaphoreType.DMA,
            pltpu.SemaphoreType.DMA,
            pltpu.SemaphoreType.DMA,
            pltpu.SemaphoreType.DMA,
            pltpu.SemaphoreType.DMA,
            pltpu.SemaphoreType.DMA,
            pltpu.SemaphoreType.DMA,
            pltpu.SemaphoreType.DMA,
        ],
    )
    def scat(h_hbm, g0_hbm, s0_hbm, w0_hbm, g1_hbm, s1_hbm, w1_hbm,
             z_hbm, out_hbm,
             g_v, s_v, w_v, r0, r1, r2, r3, acc_sh,
             sg0, sg1, sg2, sg3, ss0, ss1, ss2, ss3):
        cid = lax.axis_index("c")
        sid = lax.axis_index("s")
        bufs = [r0, r1, r2, r3]
        sg = [sg0, sg1, sg2, sg3]
        ss = [ss0, ss1, ss2, ss3]
        nb = jnp.where(cid == 0, nb0, nb1)

        @pl.when(cid == 0)
        def _():
            pltpu.sync_copy(g0_hbm.at[sid], g_v)

        @pl.when(cid == 1)
        def _():
            pltpu.sync_copy(g1_hbm.at[sid], g_v.at[pl.ds(0, nb1)])

        def start_gather(i, k):
            pltpu.async_copy(h_hbm.at[g_v.at[i]], bufs[k], sg[k])

        start_gather(0, 0)
        start_gather(1, 1)

        @pl.when(cid == 0)
        def _():
            pltpu.sync_copy(s0_hbm.at[sid], s_v)
            pltpu.sync_copy(w0_hbm.at[sid], w_v)

        @pl.when(cid == 1)
        def _():
            pltpu.sync_copy(s1_hbm.at[sid], s_v.at[pl.ds(0, nb1)])
            pltpu.sync_copy(w1_hbm.at[sid], w_v.at[pl.ds(0, nb1 * b)])

        pltpu.sync_copy(z_hbm.at[pl.ds(sid * rs, rs)],
                        acc_sh.at[pl.ds(sid * rs, rs)])
        if tail:
            @pl.when(sid == NS - 1)
            def _():
                pltpu.sync_copy(z_hbm.at[pl.ds(NS * rs, tail)],
                                acc_sh.at[pl.ds(NS * rs, tail)])
        plsc.subcore_barrier()

        def wait_gather(k):
            pltpu.make_async_copy(h_hbm.at[g_v.at[0]], bufs[k], sg[k]).wait()

        def start_scatter(i, k):
            pltpu.async_copy(bufs[k], acc_sh.at[s_v.at[i]], ss[k], add=True)

        def wait_scatter(k):
            pltpu.make_async_copy(bufs[k], acc_sh.at[s_v.at[0]],
                                  ss[k]).wait()

        def scale(i, k):
            rows = bufs[k]

            @plsc.parallel_loop(0, b // 16)
            def _grp(g):
                wch = w_v[pl.ds(i * b + g * 16, 16)]
                for jb in range(4):
                    es = g * 16 + jb * 4
                    prods = []
                    for j in range(4):
                        w16 = jnp.full((16,), wch[jb * 4 + j])
                        for c in range(nch):
                            prods.append(
                                rows[es + j, pl.ds(c * 16, 16)] * w16)
                    t = 0
                    for j in range(4):
                        for c in range(nch):
                            rows[es + j, pl.ds(c * 16, 16)] = prods[t]
                            t += 1

        for k in range(4):
            if k >= 2:
                wait_scatter((k + 2) % 4)
            start_gather(k + 2, (k + 2) % 4)
            wait_gather(k)
            scale(k, k)
            start_scatter(k, k)

        @pl.loop(1, nb // 4 - 1)
        def _outer(o):
            i0 = o * 4
            for k in range(4):
                wait_scatter((k + 2) % 4)
                start_gather(i0 + k + 2, (k + 2) % 4)
                wait_gather(k)
                scale(i0 + k, k)
                start_scatter(i0 + k, k)

        i0 = nb - 4
        for k in range(4):
            wait_scatter((k + 2) % 4)
            if k < 2:
                start_gather(i0 + k + 2, (k + 2) % 4)
            wait_gather(k)
            scale(i0 + k, k)
            start_scatter(i0 + k, k)
        wait_scatter(2)
        wait_scatter(3)

        plsc.subcore_barrier()
        pltpu.sync_copy(acc_sh.at[pl.ds(sid * rs, rs)],
                        out_hbm.at[cid, pl.ds(sid * rs, rs)])
        if tail:
            @pl.when(sid == NS - 1)
            def _():
                pltpu.sync_copy(acc_sh.at[pl.ds(NS * rs, tail)],
                                out_hbm.at[cid, pl.ds(NS * rs, tail)])

    return scat(h, g0, s0, w0, g1, s1, w1, zeros)


def kernel(x, edge_index, edge_weight, params):
    n, d = x.shape
    eg = edge_weight.shape[0]
    p = params

    b = 128
    nb0, nb1 = 128, 32
    cap = NS * (nb0 + nb1) * b
    assert cap >= eg and n % NS == 0
    pad = cap - eg
    cut = NS * nb0 * b

    def split(a):
        a = jnp.concatenate([a, jnp.zeros((pad,), a.dtype)])
        return a[:cut].reshape(NS, nb0, b), a[cut:].reshape(NS, nb1, b)

    src0, src1 = split(edge_index[0])
    dst0, dst1 = split(edge_index[1])
    w0, w1 = split(edge_weight)
    w0 = w0.reshape(NS, nb0 * b)
    w1 = w1.reshape(NS, nb1 * b)
    zeros = jnp.zeros((n, p['v2e0_eW2'].shape[1]), jnp.float32)

    r1 = lambda v: v.reshape(1, -1)
    rows = 1000

    def wset(name):
        return (p[name + '_eW1'], r1(p[name + '_eb1']),
                p[name + '_eW2'], r1(p[name + '_eb2']),
                p[name + '_dW1'], r1(p[name + '_db1']),
                p[name + '_dW2'], r1(p[name + '_db2']))

    v2e0 = wset('v2e0'); e2v0 = wset('e2v0')
    v2e1 = wset('v2e1'); e2v1 = wset('e2v1')

    h = _tc_first(x, *v2e0[:4], rows=rows)
    pp = _sc_scatter(h, src0, dst0, w0, src1, dst1, w1, zeros, b=b)
    h = _tc_mid(pp, *v2e0[4:], *e2v0[:4], rows=rows)
    pp = _sc_scatter(h, dst0, src0, w0, dst1, src1, w1, zeros, b=b)
    h = _tc_mid(pp, *e2v0[4:], *v2e1[:4], rows=rows)
    pp = _sc_scatter(h, src0, dst0, w0, src1, dst1, w1, zeros, b=b)
    h = _tc_mid(pp, *v2e1[4:], *e2v1[:4], rows=rows)
    pp = _sc_scatter(h, dst0, src0, w0, dst1, src1, w1, zeros, b=b)
    out = _tc_final(pp, *e2v1[4:],
                    p['clf_W1'], r1(p['clf_b1']),
                    p['clf_W2'], r1(p['clf_b2']), rows=rows)
    return out


# speedup vs baseline: 1.1203x; 1.1203x over previous
import functools

import jax
import jax.numpy as jnp
from jax import lax
from jax.experimental import pallas as pl
from jax.experimental.pallas import tpu as pltpu
from jax.experimental.pallas import tpu_sc as plsc

NC = 2
NS = 16
NW = NC * NS


def _lnorm(h, eps=1e-5):
    m = jnp.mean(h, axis=-1, keepdims=True)
    v = jnp.mean((h - m) ** 2, axis=-1, keepdims=True)
    return (h - m) * jax.lax.rsqrt(v + eps)


def _mlp_relu(h, w1, b1, w2, b2):
    h = _lnorm(h)
    h = jax.nn.relu(jnp.dot(h, w1, preferred_element_type=jnp.float32) + b1)
    h = _lnorm(h)
    h = jnp.dot(h, w2, preferred_element_type=jnp.float32) + b2
    return jax.nn.relu(h)


def _tc_first(x, w1, b1, w2, b2, *, rows):
    n, d = x.shape
    h = w1.shape[1]

    def body(x_ref, w1_ref, b1_ref, w2_ref, b2_ref, o_ref):
        o_ref[...] = _mlp_relu(x_ref[...], w1_ref[...], b1_ref[...],
                               w2_ref[...], b2_ref[...])

    full = lambda a: pl.BlockSpec(a.shape, lambda i: (0,) * a.ndim)
    return pl.pallas_call(
        body,
        grid=(n // rows,),
        in_specs=[pl.BlockSpec((rows, d), lambda i: (i, 0)),
                  full(w1), full(b1), full(w2), full(b2)],
        out_specs=pl.BlockSpec((rows, h), lambda i: (i, 0)),
        out_shape=jax.ShapeDtypeStruct((n, h), jnp.float32),
    )(x, w1, b1, w2, b2)


def _tc_mid(p, dw1, db1, dw2, db2, ew1, eb1, ew2, eb2, *, rows):
    _, n, h = p.shape

    def body(p_ref, dw1_ref, db1_ref, dw2_ref, db2_ref,
             ew1_ref, eb1_ref, ew2_ref, eb2_ref, o_ref):
        agg = p_ref[0]
        for t in range(1, p_ref.shape[0]):
            agg = agg + p_ref[t]
        hmid = _mlp_relu(agg, dw1_ref[...], db1_ref[...],
                         dw2_ref[...], db2_ref[...])
        o_ref[...] = _mlp_relu(hmid, ew1_ref[...], eb1_ref[...],
                               ew2_ref[...], eb2_ref[...])

    full = lambda a: pl.BlockSpec(a.shape, lambda i: (0,) * a.ndim)
    ws = [dw1, db1, dw2, db2, ew1, eb1, ew2, eb2]
    return pl.pallas_call(
        body,
        grid=(n // rows,),
        in_specs=[pl.BlockSpec((2, rows, h), lambda i: (0, i, 0))]
                 + [full(a) for a in ws],
        out_specs=pl.BlockSpec((rows, h), lambda i: (i, 0)),
        out_shape=jax.ShapeDtypeStruct((n, h), jnp.float32),
    )(p, *ws)


def _tc_final(p, dw1, db1, dw2, db2, cw1, cb1, cw2, cb2, *, rows):
    _, n, h = p.shape
    c = cw2.shape[1]

    def body(p_ref, dw1_ref, db1_ref, dw2_ref, db2_ref,
             cw1_ref, cb1_ref, cw2_ref, cb2_ref, o_ref):
        agg = p_ref[0]
        for t in range(1, p_ref.shape[0]):
            agg = agg + p_ref[t]
        hm = _mlp_relu(agg, dw1_ref[...], db1_ref[...],
                       dw2_ref[...], db2_ref[...])
        hm = jax.nn.relu(
            jnp.dot(hm, cw1_ref[...], preferred_element_type=jnp.float32)
            + cb1_ref[...])
        hm = _lnorm(hm)
        o_ref[...] = (jnp.dot(hm, cw2_ref[...],
                              preferred_element_type=jnp.float32)
                      + cb2_ref[...])

    full = lambda a: pl.BlockSpec(a.shape, lambda i: (0,) * a.ndim)
    ws = [dw1, db1, dw2, db2, cw1, cb1, cw2, cb2]
    return pl.pallas_call(
        body,
        grid=(n // rows,),
        in_specs=[pl.BlockSpec((2, rows, h), lambda i: (0, i, 0))]
                 + [full(a) for a in ws],
        out_specs=pl.BlockSpec((rows, c), lambda i: (i, 0)),
        out_shape=jax.ShapeDtypeStruct((n, c), jnp.float32),
    )(p, *ws)


@functools.partial(jax.jit, static_argnames=("b",))
def _sc_scatter(h, g0, s0, w0, g1, s1, w1, zeros, *, b):
    n, hdim = h.shape
    nb0 = g0.shape[1]
    nb1 = g1.shape[1]
    nch = hdim // 16
    assert nb0 % 4 == 0 and nb1 % 4 == 0 and nb1 >= 8 and b % 16 == 0
    rs = (n // NS) // 8 * 8
    tail = n - NS * rs

    mesh = plsc.VectorSubcoreMesh(core_axis_name="c", subcore_axis_name="s",
                                  num_cores=NC)

    @functools.partial(
        pl.kernel,
        mesh=mesh,
        compiler_params=pltpu.CompilerParams(use_tc_tiling_on_sc=False),
        out_type=jax.ShapeDtypeStruct((NC, n, hdim), jnp.float32),
        scratch_types=[
            pltpu.VMEM((nb0, b), jnp.int32),
            pltpu.VMEM((nb0, b), jnp.int32),
            pltpu.VMEM((nb0 * b,), jnp.float32),
            pltpu.VMEM((4, b, hdim), jnp.float32),
            pltpu.VMEM_SHARED((n, hdim), jnp.float32),
            pltpu.SemaphoreType.DMA((4,)),
            pltpu.SemaphoreType.DMA((4,)),
        ],
    )
    def scat(h_hbm, g0_hbm, s0_hbm, w0_hbm, g1_hbm, s1_hbm, w1_hbm,
             z_hbm, out_hbm,
             g_v, s_v, w_v, rbuf, acc_sh, sg, ss):
        cid = lax.axis_index("c")
        sid = lax.axis_index("s")
        nb = jnp.where(cid == 0, nb0, nb1)

        @pl.when(cid == 0)
        def _():
            pltpu.sync_copy(g0_hbm.at[sid], g_v)

        @pl.when(cid == 1)
        def _():
            pltpu.sync_copy(g1_hbm.at[sid], g_v.at[pl.ds(0, nb1)])

        def start_gather(i, k):
            pltpu.async_copy(h_hbm.at[g_v.at[i]], rbuf.at[k], sg.at[k])

        start_gather(0, 0)
        start_gather(1, 1)

        @pl.when(cid == 0)
        def _():
            pltpu.sync_copy(s0_hbm.at[sid], s_v)
            pltpu.sync_copy(w0_hbm.at[sid], w_v)

        @pl.when(cid == 1)
        def _():
            pltpu.sync_copy(s1_hbm.at[sid], s_v.at[pl.ds(0, nb1)])
            pltpu.sync_copy(w1_hbm.at[sid], w_v.at[pl.ds(0, nb1 * b)])

        pltpu.sync_copy(z_hbm.at[pl.ds(sid * rs, rs)],
                        acc_sh.at[pl.ds(sid * rs, rs)])
        if tail:
            @pl.when(sid == NS - 1)
            def _():
                pltpu.sync_copy(z_hbm.at[pl.ds(NS * rs, tail)],
                                acc_sh.at[pl.ds(NS * rs, tail)])
        plsc.subcore_barrier()

        def wait_gather(k):
            pltpu.make_async_copy(h_hbm.at[g_v.at[0]], rbuf.at[k],
                                  sg.at[k]).wait()

        def start_scatter(i, k):
            pltpu.async_copy(rbuf.at[k], acc_sh.at[s_v.at[i]], ss.at[k],
                             add=True)

        def wait_scatter(k):
            pltpu.make_async_copy(rbuf.at[k], acc_sh.at[s_v.at[0]],
                                  ss.at[k]).wait()

        def scale(i, k):
            rows = rbuf.at[k]

            @plsc.parallel_loop(0, b // 16)
            def _grp(g):
                wch = w_v[pl.ds(i * b + g * 16, 16)]
                for jb in range(4):
                    es = g * 16 + jb * 4
                    prods = []
                    for j in range(4):
                        w16 = jnp.full((16,), wch[jb * 4 + j])
                        for c in range(nch):
                            prods.append(
                                rows[es + j, pl.ds(c * 16, 16)] * w16)
                    t = 0
                    for j in range(4):
                        for c in range(nch):
                            rows[es + j, pl.ds(c * 16, 16)] = prods[t]
                            t += 1

        @pl.loop(0, nb)
        def _batch(i):
            k = lax.rem(i, 4)

            @pl.when(i >= 2)
            def _():
                wait_scatter(lax.rem(i + 2, 4))

            @pl.when(i + 2 < nb)
            def _():
                start_gather(i + 2, lax.rem(i + 2, 4))

            wait_gather(k)
            scale(i, k)
            start_scatter(i, k)

        wait_scatter(lax.rem(nb - 2, 4))
        wait_scatter(lax.rem(nb - 1, 4))

        plsc.subcore_barrier()
        pltpu.sync_copy(acc_sh.at[pl.ds(sid * rs, rs)],
                        out_hbm.at[cid, pl.ds(sid * rs, rs)])
        if tail:
            @pl.when(sid == NS - 1)
            def _():
                pltpu.sync_copy(acc_sh.at[pl.ds(NS * rs, tail)],
                                out_hbm.at[cid, pl.ds(NS * rs, tail)])

    return scat(h, g0, s0, w0, g1, s1, w1, zeros)


def kernel(x, edge_index, edge_weight, params):
    n, d = x.shape
    eg = edge_weight.shape[0]
    p = params

    b = 128
    nb0, nb1 = 148, 12
    cap = NS * (nb0 + nb1) * b
    assert cap >= eg and n % NS == 0
    pad = cap - eg
    cut = NS * nb0 * b

    def split(a):
        a = jnp.concatenate([a, jnp.zeros((pad,), a.dtype)])
        return a[:cut].reshape(NS, nb0, b), a[cut:].reshape(NS, nb1, b)

    src0, src1 = split(edge_index[0])
    dst0, dst1 = split(edge_index[1])
    w0, w1 = split(edge_weight)
    w0 = w0.reshape(NS, nb0 * b)
    w1 = w1.reshape(NS, nb1 * b)
    zeros = jnp.zeros((n, p['v2e0_eW2'].shape[1]), jnp.float32)

    r1 = lambda v: v.reshape(1, -1)
    rows = 1000

    def wset(name):
        return (p[name + '_eW1'], r1(p[name + '_eb1']),
                p[name + '_eW2'], r1(p[name + '_eb2']),
                p[name + '_dW1'], r1(p[name + '_db1']),
                p[name + '_dW2'], r1(p[name + '_db2']))

    v2e0 = wset('v2e0'); e2v0 = wset('e2v0')
    v2e1 = wset('v2e1'); e2v1 = wset('e2v1')

    h = _tc_first(x, *v2e0[:4], rows=rows)
    pp = _sc_scatter(h, src0, dst0, w0, src1, dst1, w1, zeros, b=b)
    h = _tc_mid(pp, *v2e0[4:], *e2v0[:4], rows=rows)
    pp = _sc_scatter(h, dst0, src0, w0, dst1, src1, w1, zeros, b=b)
    h = _tc_mid(pp, *e2v0[4:], *v2e1[:4], rows=rows)
    pp = _sc_scatter(h, src0, dst0, w0, src1, dst1, w1, zeros, b=b)
    h = _tc_mid(pp, *v2e1[4:], *e2v1[:4], rows=rows)
    pp = _sc_scatter(h, dst0, src0, w0, dst1, src1, w1, zeros, b=b)
    out = _tc_final(pp, *e2v1[4:],
                    p['clf_W1'], r1(p['clf_b1']),
                    p['clf_W2'], r1(p['clf_b2']), rows=rows)
    return out
